# baseline (speedup 1.0000x reference)
import jax
import jax.numpy as jnp
import numpy as np
from jax.experimental import pallas as pl
from jax.experimental.pallas import tpu as pltpu

BT = 2048
C = 8
XW = 32
PW = 18


def _net_kernel(x_ref, a1_ref, b1_ref, a2_ref, b2_ref,
                wl1_ref, bl1_ref, wl2_ref, bl2_ref,
                out_ref,
                xs_ref, p_ref, flat_ref):
    f32 = jnp.float32

    bf16 = jnp.bfloat16
    z4 = jnp.zeros((4, BT), bf16)
    for r in range(28):
        xs_ref[XW * r:XW * r + 28] = x_ref[r].astype(bf16)
        xs_ref[XW * r + 28:XW * (r + 1)] = z4

    zrow = jnp.zeros((2, PW, C, BT), bf16)
    p_ref[0:2] = zrow
    p_ref[PW - 2:PW] = zrow
    zcol = jnp.zeros((PW, 2, C, BT), bf16)
    p_ref[:, 0:2] = zcol
    p_ref[:, PW - 2:PW] = zcol

    for s in range(14):
        if s == 0:
            start, v = 0, 0
        elif s == 13:
            start, v = 24 * XW, 2
        else:
            start, v = (2 * s - 1) * XW, 1
        x4 = xs_ref[start:start + 4 * XW]
        z = jnp.dot(a1_ref[v], x4, preferred_element_type=f32)
        z4 = z.reshape(2, 28, C, BT)
        m = jnp.maximum(z4[0], z4[1]).reshape(14, 2, C, BT)
        row = jnp.maximum(m[:, 0], m[:, 1])
        row = jnp.maximum(row + b1_ref[...], 0.0)
        p_ref[s + 2, 2:16] = row.astype(bf16)

    for h in range(7):
        xa = p_ref[2 * h:2 * h + 5].reshape(5 * PW * C, BT)
        xb = p_ref[2 * h + 1:2 * h + 6].reshape(5 * PW * C, BT)
        za = jnp.dot(a2_ref[...], xa, preferred_element_type=f32)
        zb = jnp.dot(a2_ref[...], xb, preferred_element_type=f32)
        m = jnp.maximum(za, zb)[:112].reshape(7, 2, C, BT)
        pooled = jnp.maximum(m[:, 0], m[:, 1])
        pooled = jnp.maximum(pooled + b2_ref[...], 0.0)
        flat_ref[h] = pooled.reshape(7 * C, BT)

    flat = flat_ref[...].reshape(7 * 7 * C, BT)
    h1 = jnp.dot(wl1_ref[...], flat, preferred_element_type=f32)
    h1 = jnp.maximum(h1 + bl1_ref[...], 0.0)
    h2 = jnp.dot(wl2_ref[...], h1, preferred_element_type=f32)
    out_ref[...] = jnp.maximum(h2 + bl2_ref[...], 0.0)


def _prep_params(w1, b1, w2, b2, wl1, bl1, wl2, bl2):
    f32 = jnp.float32

    rr = np.arange(2)
    ky = np.arange(3)
    kx = np.arange(3)
    x28 = np.arange(28)
    cc = x28[None, :, None] + kx[:, None, None] - 1
    Ec = (cc == np.arange(XW)[None, None, :]) & (np.arange(XW)[None, None, :] <= 27)
    Ec = jnp.asarray(Ec, f32)
    w1f = w1[:, 0].astype(f32)
    a1s = []
    for dv in (-1, 0, 1):
        R = (rr[:, None, None] + ky[None, :, None] + dv
             == np.arange(4)[None, None, :])
        a1v = jnp.einsum('oyk,qyr,kxc->qxorc', w1f, jnp.asarray(R, f32), Ec)
        a1s.append(a1v.reshape(2 * 28 * C, 4 * XW))
    a1 = jnp.stack(a1s).astype(jnp.bfloat16)
    b1k = jnp.broadcast_to(b1.astype(f32)[:, None], (C, BT))

    kx5 = np.arange(5)
    x14 = np.arange(14)
    E2 = (x14[None, :, None] + kx5[:, None, None] == np.arange(18)[None, None, :])
    E2 = jnp.asarray(E2, f32)
    a2 = jnp.einsum('oiyk,kxc->xoyci', w2.astype(f32), E2)
    a2 = a2.reshape(14 * C, 5 * PW * C)
    a2 = jnp.pad(a2, ((0, 16), (0, 0))).astype(jnp.bfloat16)
    b2k = jnp.broadcast_to(b2.astype(f32)[:, None], (C, BT))

    r = jnp.arange(392)
    pos, c = r // C, r % C
    perm = c * 49 + pos
    wl1k = wl1[:, perm].astype(f32)
    bl1k = jnp.broadcast_to(bl1.astype(f32)[:, None], (128, BT))

    wl2k = jnp.zeros((16, 128), f32).at[:10].set(wl2.astype(f32))
    bl2k = jnp.broadcast_to(
        jnp.zeros((16,), f32).at[:10].set(bl2.astype(f32))[:, None], (16, BT))
    return a1, b1k, a2, b2k, wl1k, bl1k, wl2k, bl2k


def kernel(x, w1, b1, w2, b2, wl1, bl1, wl2, bl2):
    xf = x.astype(jnp.float32)
    n = xf.shape[0]
    n_pad = ((n + BT - 1) // BT) * BT
    grid_n = n_pad // BT

    xt = jnp.transpose(xf[:, 0], (1, 2, 0))
    if n_pad != n:
        xt = jnp.pad(xt, ((0, 0), (0, 0), (0, n_pad - n)))

    a1, b1k, a2, b2k, wl1k, bl1k, wl2k, bl2k = _prep_params(
        w1, b1, w2, b2, wl1, bl1, wl2, bl2)

    def _resident(a):
        nd = a.ndim
        return pl.BlockSpec(a.shape, lambda i, _nd=nd: (0,) * _nd)

    out = pl.pallas_call(
        _net_kernel,
        out_shape=jax.ShapeDtypeStruct((16, n_pad), jnp.float32),
        grid=(grid_n,),
        in_specs=[
            pl.BlockSpec((28, 28, BT), lambda i: (0, 0, i)),
            _resident(a1), _resident(b1k),
            _resident(a2), _resident(b2k),
            _resident(wl1k), _resident(bl1k),
            _resident(wl2k), _resident(bl2k),
        ],
        out_specs=pl.BlockSpec((16, BT), lambda i: (0, i)),
        scratch_shapes=[
            pltpu.VMEM((28 * XW, BT), jnp.bfloat16),
            pltpu.VMEM((PW, PW, C, BT), jnp.bfloat16),
            pltpu.VMEM((7, 7 * C, BT), jnp.float32),
        ],
        compiler_params=pltpu.CompilerParams(
            dimension_semantics=("parallel",),
            vmem_limit_bytes=64 * 1024 * 1024,
        ),
    )(xt, a1, b1k, a2, b2k, wl1k, bl1k, wl2k, bl2k)

    return jnp.transpose(out[:10, :n])
